# baseline (speedup 1.0000x reference)
import functools

import jax
import jax.numpy as jnp
from jax import lax
from jax.experimental import pallas as pl
from jax.experimental.pallas import tpu as pltpu
from jax.experimental.pallas import tpu_sc as plsc

H = 128
T_SEQ = 50
TP = 56
B_TOT = 4096
BB = 128
NW = 32
CH = 128
SEQ_CHUNKS = B_TOT * TP // (NW * CH)
BPW = B_TOT // NW


def _sc_gather_body(idx_hbm, src_iid_hbm, src_uid_hbm, tgt_iid_hbm,
                    ufea_out, uid_out, iid_out,
                    idx_v, buf0, buf1, sem0, sem1):
    wid = lax.axis_index("s") * 2 + lax.axis_index("c")
    pltpu.sync_copy(idx_hbm.at[wid], idx_v)

    base = wid * (SEQ_CHUNKS * CH)

    def _wait(buf, sem):
        pltpu.make_async_copy(src_iid_hbm.at[idx_v.at[0]], buf, sem).wait()

    pltpu.async_copy(src_iid_hbm.at[idx_v.at[0]], buf0, sem0)
    pltpu.async_copy(src_iid_hbm.at[idx_v.at[1]], buf1, sem1)

    def outer(g, carry):
        for par, buf, sem in ((0, buf0, sem0), (1, buf1, sem1)):
            j = 2 * g + par
            _wait(buf, sem)
            pltpu.sync_copy(buf, ufea_out.at[pl.ds(base + j * CH, CH)])

            @pl.when(j + 2 < SEQ_CHUNKS)
            def _():
                pltpu.async_copy(src_iid_hbm.at[idx_v.at[j + 2]], buf, sem)
        return carry

    lax.fori_loop(0, SEQ_CHUNKS // 2, outer, 0)

    ub = wid * BPW
    pltpu.async_copy(src_uid_hbm.at[idx_v.at[SEQ_CHUNKS]], buf0, sem0)
    pltpu.async_copy(tgt_iid_hbm.at[idx_v.at[SEQ_CHUNKS + 1]], buf1, sem1)
    _wait(buf0, sem0)
    pltpu.sync_copy(buf0, uid_out.at[pl.ds(ub, BPW)])
    _wait(buf1, sem1)
    pltpu.sync_copy(buf1, iid_out.at[pl.ds(ub, BPW)])


_sc_gather = functools.partial(
    pl.kernel,
    out_type=[
        jax.ShapeDtypeStruct((B_TOT * TP, H), jnp.float32),
        jax.ShapeDtypeStruct((B_TOT, H), jnp.float32),
        jax.ShapeDtypeStruct((B_TOT, H), jnp.float32),
    ],
    mesh=plsc.VectorSubcoreMesh(core_axis_name="c", subcore_axis_name="s",
                                num_cores=2, num_subcores=16),
    scratch_types=[
        pltpu.VMEM((SEQ_CHUNKS + 2, CH), jnp.int32),
        pltpu.VMEM((CH, H), jnp.float32),
        pltpu.VMEM((CH, H), jnp.float32),
        pltpu.SemaphoreType.DMA,
        pltpu.SemaphoreType.DMA,
    ],
)(_sc_gather_body)


def _tc_body(seq3_ref, ufea_ref, u_ref, iid_ref, kw1_ref, kb1_ref, kw2_ref,
             dw1_ref, db1_ref, w2_ref, db2m_ref, linw_ref,
             out_ref, loss_ref, acc_ref):
    i = pl.program_id(0)
    u2 = ufea_ref[...]
    h = jnp.maximum(
        lax.dot_general(u2, kw1_ref[...], (((1,), (1,)), ((), ())),
                        preferred_element_type=jnp.float32)
        + kb1_ref[...], 0.0)
    s = h * kw2_ref[...]
    ek3 = jnp.sum(s.reshape(BB, TP, H), axis=2, keepdims=True)
    mask3 = (seq3_ref[...] == 0).astype(jnp.float32)
    t3 = ek3 - mask3 * 1e8
    m3 = jnp.max(t3, axis=1, keepdims=True)
    e3 = jnp.exp(t3 - m3)
    att3 = e3 / jnp.sum(e3, axis=1, keepdims=True)
    his = jnp.sum(att3 * u2.reshape(BB, TP, H), axis=1)
    dec = jnp.maximum(
        lax.dot_general(his, dw1_ref[...], (((1,), (1,)), ((), ())),
                        preferred_element_type=jnp.float32)
        + db1_ref[...], 0.0)
    u = u_ref[...]
    c = lax.dot_general(u, w2_ref[...], (((1,), (0,)), ((), ())),
                        preferred_element_type=jnp.float32)
    uid = lax.dot_general(u, db2m_ref[...], (((1,), (0,)), ((), ())),
                          preferred_element_type=jnp.float32)
    for m in range(64):
        uid = uid + dec[:, m:m + 1] * c[:, m * H:(m + 1) * H]
    iid = iid_ref[...]
    out_ref[...] = jnp.sum(uid * iid * linw_ref[...], axis=1, keepdims=True)
    sq = jnp.sum(uid * uid) + jnp.sum(iid * iid)
    prev = jnp.where(i == 0, 0.0, acc_ref[0])
    tot = prev + sq
    acc_ref[0] = tot

    @pl.when(i == pl.num_programs(0) - 1)
    def _():
        loss_ref[...] = jnp.full((1, 1), jnp.sqrt(tot) / B_TOT, jnp.float32)


_GRID = B_TOT // BB

_tc_call = pl.pallas_call(
    _tc_body,
    grid=(_GRID,),
    in_specs=[
        pl.BlockSpec((BB, TP, 1), lambda i: (i, 0, 0)),
        pl.BlockSpec((BB * TP, H), lambda i: (i, 0)),
        pl.BlockSpec((BB, H), lambda i: (i, 0)),
        pl.BlockSpec((BB, H), lambda i: (i, 0)),
        pl.BlockSpec((H, H), lambda i: (0, 0)),
        pl.BlockSpec((1, H), lambda i: (0, 0)),
        pl.BlockSpec((1, H), lambda i: (0, 0)),
        pl.BlockSpec((64, H), lambda i: (0, 0)),
        pl.BlockSpec((1, 64), lambda i: (0, 0)),
        pl.BlockSpec((H, 64 * H), lambda i: (0, 0)),
        pl.BlockSpec((H, H), lambda i: (0, 0)),
        pl.BlockSpec((1, H), lambda i: (0, 0)),
    ],
    out_specs=[
        pl.BlockSpec((BB, 1), lambda i: (i, 0)),
        pl.BlockSpec((1, 1), lambda i: (0, 0)),
    ],
    out_shape=[
        jax.ShapeDtypeStruct((B_TOT, 1), jnp.float32),
        jax.ShapeDtypeStruct((1, 1), jnp.float32),
    ],
    scratch_shapes=[pltpu.SMEM((1,), jnp.float32)],
)


def kernel(x, src_uid, src_iid, tgt_iid, lin_w, k_w1, k_b1, k_w2,
           d_w1, d_b1, d_w2, d_b2):
    seqp = jnp.pad(x[:, 2:], ((0, 0), (0, TP - T_SEQ)))
    seq3 = seqp.reshape(B_TOT, TP, 1)
    idx_seq = seqp.reshape(NW, SEQ_CHUNKS, CH)
    idx_uid = x[:, 0].reshape(NW, 1, BPW)
    idx_iid = x[:, 1].reshape(NW, 1, BPW)
    idx_all = jnp.concatenate([idx_seq, idx_uid, idx_iid], axis=1)

    ufea, uid_rows, iid_rows = _sc_gather(idx_all, src_iid, src_uid, tgt_iid)

    w2 = d_w2.reshape(H, H, 64).transpose(0, 2, 1).reshape(H, 64 * H)
    db2m = d_b2.reshape(H, H)

    out, loss = _tc_call(seq3, ufea, uid_rows, iid_rows,
                         k_w1, k_b1.reshape(1, H), k_w2, d_w1,
                         d_b1.reshape(1, 64), w2, db2m, lin_w)
    return out.reshape(B_TOT), loss.reshape(())
